# baseline (speedup 1.0000x reference)
import functools

import jax
import jax.numpy as jnp
from jax import lax
from jax.experimental import pallas as pl
from jax.experimental.pallas import tpu as pltpu
from jax.experimental.pallas import tpu_sc as plsc

_S = 1152
_SEG = 2048


def kernel(class_id, table):
    (B,) = class_id.shape
    V, D = table.shape
    info = plsc.get_sparse_core_info()
    NC, NS = info.num_cores, info.num_subcores
    NW = NC * NS

    v128 = (V // 128) * 128
    n_slab = v128 // _S
    n_virt = n_slab + 1
    n_seg = B // _SEG

    table_t = table.T
    tail = table[v128:]
    n_tail = V - v128
    idx2 = class_id.astype(jnp.int32).reshape(n_seg, _SEG)
    mesh = plsc.VectorSubcoreMesh(core_axis_name="c", subcore_axis_name="s")

    @functools.partial(
        pl.kernel,
        mesh=mesh,
        out_type=jax.ShapeDtypeStruct((B, D), jnp.float32),
        compiler_params=pltpu.CompilerParams(needs_layout_passes=False),
        scratch_types=[
            pltpu.VMEM((2, D, _S), jnp.float32),
            pltpu.VMEM((_SEG,), jnp.int32),
            pltpu.VMEM((B,), jnp.int32),
            pltpu.VMEM((B,), jnp.int32),
            pltpu.VMEM((n_tail, D), jnp.float32),
            pltpu.VMEM((16, D), jnp.float32),
            pltpu.SemaphoreType.DMA,
            pltpu.SemaphoreType.DMA,
        ],
    )
    def emb(
        table_hbm,
        idx_hbm,
        tail_hbm,
        out_hbm,
        slab2,
        qseg_v,
        midx_v,
        mpos_v,
        tail_v,
        row_v,
        slab_sem,
        out_sem,
    ):
        cid = lax.axis_index("c")
        sid = lax.axis_index("s")
        wid = sid * NC + cid
        my_n = n_virt // NW + jnp.where(wid < n_virt % NW, 1, 0)

        pltpu.sync_copy(tail_hbm, tail_v)

        lane16 = lax.iota(jnp.int32, 16)
        jcol = lax.iota(jnp.int32, 16)
        zeros16 = jnp.zeros((16,), jnp.int32)

        cnt = 0
        for seg in range(n_seg):
            pltpu.sync_copy(idx_hbm.at[seg], qseg_v)

            def g_body(g, c, seg=seg):
                vec = qseg_v[pl.ds(g * 16, 16)]
                mine = (vec // _S) % NW == wid
                plsc.store_compressed(
                    midx_v.at[pl.ds(c, 16)], vec, mask=mine
                )
                pos = seg * _SEG + g * 16 + lane16
                plsc.store_compressed(
                    mpos_v.at[pl.ds(c, 16)], pos, mask=mine
                )
                npop = plsc.all_reduce_population_count(mine)
                return c + npop[0]

            cnt = lax.fori_loop(0, _SEG // 16, g_body, cnt)

        def start_slab(s, buf):
            pltpu.make_async_copy(
                table_hbm.at[:, pl.ds(s * _S, _S)], slab2.at[buf], slab_sem
            ).start()

        def wait_slab(buf):
            pltpu.make_async_copy(
                table_hbm.at[:, pl.ds(0, _S)], slab2.at[buf], slab_sem
            ).wait()

        start_slab(wid, 0)

        def slab_body(sl, carry):
            s = wid + NW * sl
            buf = sl % 2
            is_tail = s == n_slab

            @pl.when(jnp.logical_not(is_tail))
            def _():
                wait_slab(buf)

            s_next = s + NW

            @pl.when((sl + 1 < my_n) & (s_next < n_slab))
            def _():
                start_slab(s_next, 1 - buf)

            def h_body(h, carry2):
                mvec = midx_v[pl.ds(h * 16, 16)]
                pvec = mpos_v[pl.ds(h * 16, 16)]
                for k in range(16):
                    mvk = mvec[k]
                    cond = ((h * 16 + k) < cnt) & (mvk // _S == s)

                    @pl.when(cond)
                    def _(k=k, mvk=mvk, pvec=pvec, buf=buf, is_tail=is_tail):
                        lane_s = mvk - s * _S
                        lo_s = plsc.load_gather(
                            slab2, [zeros16 + buf, jcol, zeros16 + lane_s]
                        )
                        hi_s = plsc.load_gather(
                            slab2, [zeros16 + buf, jcol + 16, zeros16 + lane_s]
                        )
                        lane_t = jnp.clip(mvk - v128, 0, n_tail - 1)
                        lo_t = plsc.load_gather(
                            tail_v, [zeros16 + lane_t, jcol]
                        )
                        hi_t = plsc.load_gather(
                            tail_v, [zeros16 + lane_t, jcol + 16]
                        )
                        row_v[k, pl.ds(0, 16)] = jnp.where(is_tail, lo_t, lo_s)
                        row_v[k, pl.ds(16, 16)] = jnp.where(
                            is_tail, hi_t, hi_s
                        )
                        pltpu.make_async_copy(
                            row_v.at[pl.ds(k, 1)],
                            out_hbm.at[pl.ds(pvec[k], 1)],
                            out_sem,
                        ).start()

                for k in range(16):
                    mvk = mvec[k]
                    cond = ((h * 16 + k) < cnt) & (mvk // _S == s)

                    @pl.when(cond)
                    def _(k=k):
                        pltpu.make_async_copy(
                            row_v.at[pl.ds(k, 1)],
                            out_hbm.at[pl.ds(0, 1)],
                            out_sem,
                        ).wait()

                return carry2

            lax.fori_loop(0, (cnt + 15) // 16, h_body, 0)
            return carry

        lax.fori_loop(0, my_n, slab_body, 0)

    return emb(table_t, idx2, tail)


# speedup vs baseline: 2.0253x; 2.0253x over previous
import functools

import jax
import jax.numpy as jnp
from jax import lax
from jax.experimental import pallas as pl
from jax.experimental.pallas import tpu as pltpu
from jax.experimental.pallas import tpu_sc as plsc

_S = 1152
_SEG = 2048


def kernel(class_id, table):
    (B,) = class_id.shape
    V, D = table.shape
    info = plsc.get_sparse_core_info()
    NC, NS = info.num_cores, info.num_subcores
    NW = NC * NS

    v128 = (V // 128) * 128
    n_slab = v128 // _S
    n_virt = n_slab + 1
    n_seg = B // _SEG

    table_t = table.T
    tail = table[v128:]
    n_tail = V - v128
    idx2 = class_id.astype(jnp.int32).reshape(n_seg, _SEG)
    mesh = plsc.VectorSubcoreMesh(core_axis_name="c", subcore_axis_name="s")

    @functools.partial(
        pl.kernel,
        mesh=mesh,
        out_type=jax.ShapeDtypeStruct((B, D), jnp.float32),
        compiler_params=pltpu.CompilerParams(needs_layout_passes=False),
        scratch_types=[
            pltpu.VMEM((2, D, _S), jnp.float32),
            pltpu.VMEM((_SEG,), jnp.int32),
            pltpu.VMEM((B,), jnp.int32),
            pltpu.VMEM((B,), jnp.int32),
            pltpu.VMEM((n_tail, D), jnp.float32),
            pltpu.VMEM((16, D), jnp.float32),
            pltpu.VMEM((16,), jnp.int32),
            pltpu.VMEM((16,), jnp.int32),
            pltpu.SemaphoreType.DMA,
            pltpu.SemaphoreType.DMA,
        ],
    )
    def emb(
        table_hbm,
        idx_hbm,
        tail_hbm,
        out_hbm,
        slab2,
        qseg_v,
        midx_v,
        mpos_v,
        tail_v,
        row_v,
        sl_lane,
        sl_pos,
        slab_sem,
        out_sem,
    ):
        cid = lax.axis_index("c")
        sid = lax.axis_index("s")
        wid = sid * NC + cid
        my_n = n_virt // NW + jnp.where(wid < n_virt % NW, 1, 0)

        pltpu.sync_copy(tail_hbm, tail_v)

        lane16 = lax.iota(jnp.int32, 16)
        jcol = lax.iota(jnp.int32, 16)
        zeros16 = jnp.zeros((16,), jnp.int32)

        cnt = 0
        for seg in range(n_seg):
            pltpu.sync_copy(idx_hbm.at[seg], qseg_v)

            def g_body(g, c, seg=seg):
                vec = qseg_v[pl.ds(g * 16, 16)]
                mine = (vec // _S) % NW == wid
                plsc.store_compressed(
                    midx_v.at[pl.ds(c, 16)], vec, mask=mine
                )
                pos = seg * _SEG + g * 16 + lane16
                plsc.store_compressed(
                    mpos_v.at[pl.ds(c, 16)], pos, mask=mine
                )
                npop = plsc.all_reduce_population_count(mine)
                return c + npop[0]

            cnt = lax.fori_loop(0, _SEG // 16, g_body, cnt)

        def start_slab(s, buf):
            pltpu.make_async_copy(
                table_hbm.at[:, pl.ds(s * _S, _S)], slab2.at[buf], slab_sem
            ).start()

        def wait_slab(buf):
            pltpu.make_async_copy(
                table_hbm.at[:, pl.ds(0, _S)], slab2.at[buf], slab_sem
            ).wait()

        start_slab(wid, 0)

        def slab_body(sl, carry):
            s = wid + NW * sl
            buf = sl % 2
            is_tail = s == n_slab

            @pl.when(jnp.logical_not(is_tail))
            def _():
                wait_slab(buf)

            s_next = s + NW

            @pl.when((sl + 1 < my_n) & (s_next < n_slab))
            def _():
                start_slab(s_next, 1 - buf)

            def h_body(h, carry2):
                mvec = midx_v[pl.ds(h * 16, 16)]
                m2 = ((lane16 + h * 16) < cnt) & (mvec // _S == s)
                np0 = plsc.all_reduce_population_count(m2)[0]

                @pl.when(np0 > 0)
                def _(buf=buf, is_tail=is_tail):
                    pvec = mpos_v[pl.ds(h * 16, 16)]
                    plsc.store_compressed(
                        sl_lane.at[pl.ds(0, 16)], mvec - s * _S, mask=m2
                    )
                    plsc.store_compressed(
                        sl_pos.at[pl.ds(0, 16)], pvec, mask=m2
                    )
                    lv = sl_lane[pl.ds(0, 16)]
                    pv = sl_pos[pl.ds(0, 16)]
                    for k in range(16):
                        @pl.when(k < np0)
                        def _(k=k):
                            lane_s = lv[k]
                            lo_s = plsc.load_gather(
                                slab2,
                                [zeros16 + buf, jcol, zeros16 + lane_s],
                            )
                            hi_s = plsc.load_gather(
                                slab2,
                                [zeros16 + buf, jcol + 16, zeros16 + lane_s],
                            )
                            lane_t = jnp.clip(lane_s, 0, n_tail - 1)
                            lo_t = plsc.load_gather(
                                tail_v, [zeros16 + lane_t, jcol]
                            )
                            hi_t = plsc.load_gather(
                                tail_v, [zeros16 + lane_t, jcol + 16]
                            )
                            row_v[k, pl.ds(0, 16)] = jnp.where(
                                is_tail, lo_t, lo_s
                            )
                            row_v[k, pl.ds(16, 16)] = jnp.where(
                                is_tail, hi_t, hi_s
                            )
                            pltpu.make_async_copy(
                                row_v.at[pl.ds(k, 1)],
                                out_hbm.at[pl.ds(pv[k], 1)],
                                out_sem,
                            ).start()

                    for k in range(16):
                        @pl.when(k < np0)
                        def _(k=k):
                            pltpu.make_async_copy(
                                row_v.at[pl.ds(k, 1)],
                                out_hbm.at[pl.ds(0, 1)],
                                out_sem,
                            ).wait()

                return carry2

            lax.fori_loop(0, (cnt + 15) // 16, h_body, 0)
            return carry

        lax.fori_loop(0, my_n, slab_body, 0)

    return emb(table_t, idx2, tail)


# speedup vs baseline: 2.4565x; 1.2130x over previous
import functools

import jax
import jax.numpy as jnp
from jax import lax
from jax.experimental import pallas as pl
from jax.experimental.pallas import tpu as pltpu
from jax.experimental.pallas import tpu_sc as plsc

_S = 1152
_SEG = 2048


def kernel(class_id, table):
    (B,) = class_id.shape
    V, D = table.shape
    info = plsc.get_sparse_core_info()
    NC, NS = info.num_cores, info.num_subcores
    NW = NC * NS

    v128 = (V // 128) * 128
    n_slab = v128 // _S
    n_virt = n_slab + 1
    n_seg = B // _SEG

    table_t = table.T
    tail = table[v128:]
    n_tail = V - v128
    idx2 = class_id.astype(jnp.int32).reshape(n_seg, _SEG)
    mesh = plsc.VectorSubcoreMesh(core_axis_name="c", subcore_axis_name="s")

    @functools.partial(
        pl.kernel,
        mesh=mesh,
        out_type=jax.ShapeDtypeStruct((B, D), jnp.float32),
        compiler_params=pltpu.CompilerParams(needs_layout_passes=False),
        scratch_types=[
            pltpu.VMEM((2, D, _S), jnp.float32),
            pltpu.VMEM((_SEG,), jnp.int32),
            pltpu.VMEM((B,), jnp.int32),
            pltpu.VMEM((B,), jnp.int32),
            pltpu.VMEM((n_tail, D), jnp.float32),
            pltpu.VMEM((16, D), jnp.float32),
            pltpu.VMEM((16,), jnp.int32),
            pltpu.VMEM((16,), jnp.int32),
            pltpu.SemaphoreType.DMA,
            pltpu.SemaphoreType.DMA,
        ],
    )
    def emb(
        table_hbm,
        idx_hbm,
        tail_hbm,
        out_hbm,
        slab2,
        qseg_v,
        midx_v,
        mpos_v,
        tail_v,
        row_v,
        sl_lane,
        sl_pos,
        slab_sem,
        out_sem,
    ):
        cid = lax.axis_index("c")
        sid = lax.axis_index("s")
        wid = sid * NC + cid
        my_n = n_virt // NW + jnp.where(wid < n_virt % NW, 1, 0)

        pltpu.sync_copy(tail_hbm, tail_v)

        lane16 = lax.iota(jnp.int32, 16)
        jcol = lax.iota(jnp.int32, 16)
        zeros16 = jnp.zeros((16,), jnp.int32)

        cnt = 0
        for seg in range(n_seg):
            pltpu.sync_copy(idx_hbm.at[seg], qseg_v)

            def g_body(g, c, seg=seg):
                vec = qseg_v[pl.ds(g * 16, 16)]
                sl = vec // _S
                mine = (sl & (NW - 1)) == wid
                key = (sl << 12) | (vec - sl * _S)
                plsc.store_compressed(
                    midx_v.at[pl.ds(c, 16)], key, mask=mine
                )
                pos = seg * _SEG + g * 16 + lane16
                plsc.store_compressed(
                    mpos_v.at[pl.ds(c, 16)], pos, mask=mine
                )
                npop = plsc.all_reduce_population_count(mine)
                return c + npop[0]

            cnt = lax.fori_loop(0, _SEG // 16, g_body, cnt)

        def start_slab(s, buf):
            pltpu.make_async_copy(
                table_hbm.at[:, pl.ds(s * _S, _S)], slab2.at[buf], slab_sem
            ).start()

        def wait_slab(buf):
            pltpu.make_async_copy(
                table_hbm.at[:, pl.ds(0, _S)], slab2.at[buf], slab_sem
            ).wait()

        start_slab(wid, 0)

        def slab_body(sl, carry):
            s = wid + NW * sl
            buf = sl % 2
            is_tail = s == n_slab

            @pl.when(jnp.logical_not(is_tail))
            def _():
                wait_slab(buf)

            s_next = s + NW

            @pl.when((sl + 1 < my_n) & (s_next < n_slab))
            def _():
                start_slab(s_next, 1 - buf)

            def h_body(h, carry2):
                mvec = midx_v[pl.ds(h * 16, 16)]
                m2 = ((lane16 + h * 16) < cnt) & ((mvec >> 12) == s)
                np0 = plsc.all_reduce_population_count(m2)[0]

                @pl.when(np0 > 0)
                def _(buf=buf, is_tail=is_tail):
                    pvec = mpos_v[pl.ds(h * 16, 16)]
                    plsc.store_compressed(
                        sl_lane.at[pl.ds(0, 16)], mvec & 4095, mask=m2
                    )
                    plsc.store_compressed(
                        sl_pos.at[pl.ds(0, 16)], pvec, mask=m2
                    )
                    lv = sl_lane[pl.ds(0, 16)]
                    pv = sl_pos[pl.ds(0, 16)]
                    for k in range(16):
                        @pl.when(k < np0)
                        def _(k=k):
                            lane_s = lv[k]
                            lo_s = plsc.load_gather(
                                slab2,
                                [zeros16 + buf, jcol, zeros16 + lane_s],
                            )
                            hi_s = plsc.load_gather(
                                slab2,
                                [zeros16 + buf, jcol + 16, zeros16 + lane_s],
                            )
                            lane_t = jnp.clip(lane_s, 0, n_tail - 1)
                            lo_t = plsc.load_gather(
                                tail_v, [zeros16 + lane_t, jcol]
                            )
                            hi_t = plsc.load_gather(
                                tail_v, [zeros16 + lane_t, jcol + 16]
                            )
                            row_v[k, pl.ds(0, 16)] = jnp.where(
                                is_tail, lo_t, lo_s
                            )
                            row_v[k, pl.ds(16, 16)] = jnp.where(
                                is_tail, hi_t, hi_s
                            )
                            pltpu.make_async_copy(
                                row_v.at[pl.ds(k, 1)],
                                out_hbm.at[pl.ds(pv[k], 1)],
                                out_sem,
                            ).start()

                    for k in range(16):
                        @pl.when(k < np0)
                        def _(k=k):
                            pltpu.make_async_copy(
                                row_v.at[pl.ds(k, 1)],
                                out_hbm.at[pl.ds(0, 1)],
                                out_sem,
                            ).wait()

                return carry2

            lax.fori_loop(0, (cnt + 15) // 16, h_body, 0)
            return carry

        lax.fori_loop(0, my_n, slab_body, 0)

    return emb(table_t, idx2, tail)


# speedup vs baseline: 2.5212x; 1.0263x over previous
import functools

import jax
import jax.numpy as jnp
from jax import lax
from jax.experimental import pallas as pl
from jax.experimental.pallas import tpu as pltpu
from jax.experimental.pallas import tpu_sc as plsc

_S = 1152
_SEG = 2048


def kernel(class_id, table):
    (B,) = class_id.shape
    V, D = table.shape
    info = plsc.get_sparse_core_info()
    NC, NS = info.num_cores, info.num_subcores
    NW = NC * NS

    v128 = (V // 128) * 128
    n_slab = v128 // _S
    n_virt = n_slab + 1
    n_seg = B // _SEG

    table_t = table.T
    tail = table[v128:]
    n_tail = V - v128
    idx2 = class_id.astype(jnp.int32).reshape(n_seg, _SEG)
    mesh = plsc.VectorSubcoreMesh(core_axis_name="c", subcore_axis_name="s")

    @functools.partial(
        pl.kernel,
        mesh=mesh,
        out_type=jax.ShapeDtypeStruct((B, D), jnp.float32),
        compiler_params=pltpu.CompilerParams(needs_layout_passes=False),
        scratch_types=[
            pltpu.VMEM((2, D, _S), jnp.float32),
            pltpu.VMEM((_SEG,), jnp.int32),
            pltpu.VMEM((B,), jnp.int32),
            pltpu.VMEM((B,), jnp.int32),
            pltpu.VMEM((n_tail, D), jnp.float32),
            pltpu.VMEM((16, D), jnp.float32),
            pltpu.VMEM((16,), jnp.int32),
            pltpu.VMEM((16,), jnp.int32),
            pltpu.SemaphoreType.DMA,
            pltpu.SemaphoreType.DMA,
        ],
    )
    def emb(
        table_hbm,
        idx_hbm,
        tail_hbm,
        out_hbm,
        slab2,
        qseg_v,
        midx_v,
        mpos_v,
        tail_v,
        row_v,
        sl_lane,
        sl_pos,
        slab_sem,
        out_sem,
    ):
        cid = lax.axis_index("c")
        sid = lax.axis_index("s")
        wid = sid * NC + cid
        my_real = n_slab // NW + jnp.where(wid < n_slab % NW, 1, 0)

        pltpu.sync_copy(tail_hbm, tail_v)

        lane16 = lax.iota(jnp.int32, 16)
        jcol = lax.iota(jnp.int32, 16)
        zeros16 = jnp.zeros((16,), jnp.int32)

        cnt = 0
        for seg in range(n_seg):
            pltpu.sync_copy(idx_hbm.at[seg], qseg_v)

            def g_body(g, c, seg=seg):
                vec = qseg_v[pl.ds(g * 16, 16)]
                sl = vec // _S
                mine = (sl & (NW - 1)) == wid
                key = (sl << 12) | (vec - sl * _S)
                plsc.store_compressed(
                    midx_v.at[pl.ds(c, 16)], key, mask=mine
                )
                pos = seg * _SEG + g * 16 + lane16
                plsc.store_compressed(
                    mpos_v.at[pl.ds(c, 16)], pos, mask=mine
                )
                npop = plsc.all_reduce_population_count(mine)
                return c + npop[0]

            cnt = lax.fori_loop(0, _SEG // 16, g_body, cnt)

        def start_slab(s, buf):
            pltpu.make_async_copy(
                table_hbm.at[:, pl.ds(s * _S, _S)], slab2.at[buf], slab_sem
            ).start()

        def wait_slab(buf):
            pltpu.make_async_copy(
                table_hbm.at[:, pl.ds(0, _S)], slab2.at[buf], slab_sem
            ).wait()

        start_slab(wid, 0)

        def slab_body(sl, carry):
            s = wid + NW * sl
            buf = sl % 2
            wait_slab(buf)
            s_next = s + NW

            @pl.when((sl + 1 < my_real) & (s_next < n_slab))
            def _():
                start_slab(s_next, 1 - buf)

            def h_body(h, carry2):
                mvec = midx_v[pl.ds(h * 16, 16)]
                m2 = ((lane16 + h * 16) < cnt) & ((mvec >> 12) == s)
                np0 = plsc.all_reduce_population_count(m2)[0]

                @pl.when(np0 > 0)
                def _(buf=buf):
                    pvec = mpos_v[pl.ds(h * 16, 16)]
                    plsc.store_compressed(
                        sl_lane.at[pl.ds(0, 16)], mvec & 4095, mask=m2
                    )
                    plsc.store_compressed(
                        sl_pos.at[pl.ds(0, 16)], pvec, mask=m2
                    )
                    lv = sl_lane[pl.ds(0, 16)]
                    pv = sl_pos[pl.ds(0, 16)]
                    for k in range(16):
                        @pl.when(k < np0)
                        def _(k=k):
                            lane_s = lv[k]
                            lo_s = plsc.load_gather(
                                slab2,
                                [zeros16 + buf, jcol, zeros16 + lane_s],
                            )
                            hi_s = plsc.load_gather(
                                slab2,
                                [zeros16 + buf, jcol + 16, zeros16 + lane_s],
                            )
                            row_v[k, pl.ds(0, 16)] = lo_s
                            row_v[k, pl.ds(16, 16)] = hi_s
                            pltpu.make_async_copy(
                                row_v.at[pl.ds(k, 1)],
                                out_hbm.at[pl.ds(pv[k], 1)],
                                out_sem,
                            ).start()

                    for k in range(16):
                        @pl.when(k < np0)
                        def _(k=k):
                            pltpu.make_async_copy(
                                row_v.at[pl.ds(k, 1)],
                                out_hbm.at[pl.ds(0, 1)],
                                out_sem,
                            ).wait()

                return carry2

            lax.fori_loop(0, (cnt + 15) // 16, h_body, 0)
            return carry

        lax.fori_loop(0, my_real, slab_body, 0)

        @pl.when(wid == n_slab % NW)
        def _():
            def t_body(h, carry2):
                mvec = midx_v[pl.ds(h * 16, 16)]
                m2 = ((lane16 + h * 16) < cnt) & ((mvec >> 12) == n_slab)
                np0 = plsc.all_reduce_population_count(m2)[0]

                @pl.when(np0 > 0)
                def _():
                    pvec = mpos_v[pl.ds(h * 16, 16)]
                    plsc.store_compressed(
                        sl_lane.at[pl.ds(0, 16)], mvec & 4095, mask=m2
                    )
                    plsc.store_compressed(
                        sl_pos.at[pl.ds(0, 16)], pvec, mask=m2
                    )
                    lv = sl_lane[pl.ds(0, 16)]
                    pv = sl_pos[pl.ds(0, 16)]
                    for k in range(16):
                        @pl.when(k < np0)
                        def _(k=k):
                            lane_t = lv[k]
                            lo_t = plsc.load_gather(
                                tail_v, [zeros16 + lane_t, jcol]
                            )
                            hi_t = plsc.load_gather(
                                tail_v, [zeros16 + lane_t, jcol + 16]
                            )
                            row_v[k, pl.ds(0, 16)] = lo_t
                            row_v[k, pl.ds(16, 16)] = hi_t
                            pltpu.make_async_copy(
                                row_v.at[pl.ds(k, 1)],
                                out_hbm.at[pl.ds(pv[k], 1)],
                                out_sem,
                            ).start()

                    for k in range(16):
                        @pl.when(k < np0)
                        def _(k=k):
                            pltpu.make_async_copy(
                                row_v.at[pl.ds(k, 1)],
                                out_hbm.at[pl.ds(0, 1)],
                                out_sem,
                            ).wait()

                return carry2

            lax.fori_loop(0, (cnt + 15) // 16, t_body, 0)

    return emb(table_t, idx2, tail)
